# Initial kernel scaffold; baseline (speedup 1.0000x reference)
import jax
import jax.numpy as jnp
from jax.experimental import pallas as pl


def kernel(x, weight):
    raise NotImplementedError("write your pallas kernel here")


# speedup vs baseline: 1.4767x; 1.4767x over previous
import functools

import jax
import jax.numpy as jnp
from jax import lax
from jax.experimental import pallas as pl
from jax.experimental.pallas import tpu as pltpu
from jax.experimental.pallas import tpu_sc as plsc

NR_CONCEPTS = 1000000
CONCEPT_DIM = 32
BATCH = 4096
HIST = 200
NB = BATCH * HIST


def kernel(x, weight):
    info = plsc.get_sparse_core_info()
    nw = info.num_cores * info.num_subcores
    b_per_w = NB // nw
    chunk = 1600
    n_chunks = b_per_w // chunk

    mesh = plsc.VectorSubcoreMesh(core_axis_name="c", subcore_axis_name="s")

    @functools.partial(
        pl.kernel,
        mesh=mesh,
        out_type=jax.ShapeDtypeStruct((NB, CONCEPT_DIM), jnp.float32),
        scratch_types=[
            pltpu.VMEM((chunk,), jnp.int32),
            pltpu.VMEM((chunk, CONCEPT_DIM), jnp.float32),
            pltpu.SemaphoreType.DMA,
        ],
        compiler_params=pltpu.CompilerParams(use_tc_tiling_on_sc=False),
    )
    def emb_kernel(idx_hbm, table_hbm, out_hbm, idx_v, rows_v, sem):
        cid = lax.axis_index("c")
        sid = lax.axis_index("s")
        wid = sid * info.num_cores + cid
        base = wid * b_per_w

        def body(i, carry):
            off = base + i * chunk
            pltpu.sync_copy(idx_hbm.at[pl.ds(off, chunk)], idx_v)
            pltpu.async_copy(table_hbm.at[idx_v], rows_v, sem).wait()
            pltpu.sync_copy(rows_v, out_hbm.at[pl.ds(off, chunk)])
            return carry

        lax.fori_loop(0, n_chunks, body, 0)

    out = emb_kernel(x.reshape(NB), weight)
    return out.reshape(BATCH, HIST, CONCEPT_DIM)


# speedup vs baseline: 1.5003x; 1.0160x over previous
import functools

import jax
import jax.numpy as jnp
from jax import lax
from jax.experimental import pallas as pl
from jax.experimental.pallas import tpu as pltpu
from jax.experimental.pallas import tpu_sc as plsc

NR_CONCEPTS = 1000000
CONCEPT_DIM = 32
BATCH = 4096
HIST = 200
NB = BATCH * HIST


def kernel(x, weight):
    info = plsc.get_sparse_core_info()
    nw = info.num_cores * info.num_subcores
    b_per_w = NB // nw
    chunk = 1600
    n_chunks = b_per_w // chunk

    mesh = plsc.VectorSubcoreMesh(core_axis_name="c", subcore_axis_name="s")

    @functools.partial(
        pl.kernel,
        mesh=mesh,
        out_type=jax.ShapeDtypeStruct((NB, CONCEPT_DIM), jnp.float32),
        scratch_types=[
            pltpu.VMEM((b_per_w,), jnp.int32),
            pltpu.VMEM((chunk, CONCEPT_DIM), jnp.float32),
            pltpu.VMEM((chunk, CONCEPT_DIM), jnp.float32),
            pltpu.SemaphoreType.DMA,
            pltpu.SemaphoreType.DMA,
            pltpu.SemaphoreType.DMA,
            pltpu.SemaphoreType.DMA,
        ],
        compiler_params=pltpu.CompilerParams(use_tc_tiling_on_sc=False),
    )
    def emb_kernel(idx_hbm, table_hbm, out_hbm, idx_all, rows0, rows1,
                   gsem0, gsem1, wsem0, wsem1):
        cid = lax.axis_index("c")
        sid = lax.axis_index("s")
        wid = sid * info.num_cores + cid
        base = wid * b_per_w

        rows = (rows0, rows1)
        gsem = (gsem0, gsem1)
        wsem = (wsem0, wsem1)

        pltpu.sync_copy(idx_hbm.at[pl.ds(base, b_per_w)], idx_all)

        def start_gather(i, b):
            return pltpu.async_copy(
                table_hbm.at[idx_all.at[pl.ds(i * chunk, chunk)]],
                rows[b], gsem[b])

        def start_write(i, b):
            return pltpu.async_copy(
                rows[b], out_hbm.at[pl.ds(base + i * chunk, chunk)], wsem[b])

        g = [None, None]
        w = [None, None]
        g[0] = start_gather(0, 0)
        for i in range(n_chunks):
            b = i % 2
            nb = 1 - b
            if i + 1 < n_chunks:
                if w[nb] is not None:
                    w[nb].wait()
                g[nb] = start_gather(i + 1, nb)
            g[b].wait()
            w[b] = start_write(i, b)
        w[0].wait()
        w[1].wait()

    out = emb_kernel(x.reshape(NB), weight)
    return out.reshape(BATCH, HIST, CONCEPT_DIM)
